# baseline (speedup 1.0000x reference)
import functools

import jax
import jax.numpy as jnp
from jax import lax
from jax.experimental import pallas as pl
from jax.experimental.pallas import tpu as pltpu
from jax.experimental.pallas import tpu_sc as plsc

TOTAL = 1000000
NUM_EMB = TOTAL - 2
PAD = NUM_EMB - 1
START = NUM_EMB
DIM = 32
BATCH = 4096
HIST = 200

NC, NS, L = 2, 16, 16
NW = NC * NS
BB = BATCH // NW
HC = 8
NCHUNK = HIST // HC
CHUNK = BB * HC
NGRP = CHUNK // L


def _body(idx_hbm, table_hbm, be_hbm, out_hbm,
          ir0, ir1, is_v, rows_v, tv0, tv1, be_v,
          isem0, isem1, gsem, osem0, osem1):
    c = lax.axis_index("c")
    s = lax.axis_index("s")
    wid = s * NC + c
    b0 = wid * BB

    pltpu.sync_copy(be_hbm, be_v)
    lane = lax.iota(jnp.int32, L)
    rowbase = [(jnp.full((L,), bg * L, jnp.int32) + lane) * HC
               for bg in range(BB // L)]

    IR = (ir0, ir1)
    TV = (tv0, tv1)
    ISEM = (isem0, isem1)
    OSEM = (osem0, osem1)

    def idx_copy(ci, b):
        return pltpu.make_async_copy(
            idx_hbm.at[pl.ds(b0, BB), pl.ds(ci * HC, HC)], IR[b], ISEM[b])

    def out_copy(ci, b):
        return pltpu.make_async_copy(
            TV[b], out_hbm.at[pl.ds(ci * HC, HC), :, pl.ds(b0, BB)], OSEM[b])

    def pass1(b):
        def grp(g, mx):
            jv = g * L + lane
            v = plsc.load_gather(IR[b], [jv >> 3, jv & (HC - 1)])
            is_v[pl.ds(g * L, L)] = jnp.minimum(v, PAD)
            return jnp.maximum(mx, v)

        return lax.fori_loop(0, NGRP, grp, jnp.zeros((L,), jnp.int32))

    def fixup(b, mx):
        has_special = plsc.all_reduce_population_count(mx >= START)[0] > 0

        @pl.when(has_special)
        def _fix():
            def grp_body(g, carry):
                jv = g * L + lane
                v = plsc.load_gather(IR[b], [jv >> 3, jv & (HC - 1)])
                mask = v >= START
                g_has = plsc.all_reduce_population_count(mask)[0] > 0

                @pl.when(g_has)
                def _overwrite():
                    sel = jnp.clip(v - START, 0, 1)
                    for col in range(DIM):
                        colv = jnp.full((L,), col, jnp.int32)
                        repl = plsc.load_gather(be_v, [sel, colv], mask=mask)
                        plsc.store_scatter(rows_v, [jv, colv], repl,
                                           mask=mask)
                return carry

            lax.fori_loop(0, NGRP, grp_body, 0)

    def transpose(b):
        def h_body(h, carry):
            hvec = jnp.full((L,), h, jnp.int32)
            rows = [rb + h for rb in rowbase]

            def d_body(d, carry2):
                dvec = jnp.full((L,), d, jnp.int32)
                for bg in range(BB // L):
                    vals = plsc.load_gather(rows_v, [rows[bg], dvec])
                    plsc.store_scatter(
                        TV[b], [hvec, dvec,
                                jnp.full((L,), bg * L, jnp.int32) + lane],
                        vals)
                return carry2

            lax.fori_loop(0, DIM, d_body, 0)
            return carry

        lax.fori_loop(0, HC, h_body, 0)

    def stage(ci, b, wait_prev_out, fire_next_idx):
        idx_copy(ci, b).wait()
        mx = pass1(b)
        gather = pltpu.async_copy(table_hbm.at[is_v], rows_v, gsem)
        if fire_next_idx:
            idx_copy(ci + 1, 1 - b).start()
        gather.wait()
        fixup(b, mx)
        if wait_prev_out:
            out_copy(ci, b).wait()
        transpose(b)
        out_copy(ci, b).start()

    idx_copy(0, 0).start()
    stage(0, 0, False, True)
    stage(1, 1, False, True)

    def pair(p, carry):
        ci = 2 + 2 * p
        stage(ci, 0, True, True)
        stage(ci + 1, 1, True, True)
        return carry

    lax.fori_loop(0, (NCHUNK - 3) // 2, pair, 0)
    stage(NCHUNK - 1, 0, True, False)
    out_copy(NCHUNK - 2, 1).wait()
    out_copy(NCHUNK - 1, 0).wait()


@jax.jit
def _run(idxes, table, beg_end):
    f = functools.partial(
        pl.kernel,
        mesh=plsc.VectorSubcoreMesh(core_axis_name="c", subcore_axis_name="s"),
        out_type=jax.ShapeDtypeStruct((HIST, DIM, BATCH), jnp.float32),
        scratch_types=[
            pltpu.VMEM((BB, HC), jnp.int32),
            pltpu.VMEM((BB, HC), jnp.int32),
            pltpu.VMEM((CHUNK,), jnp.int32),
            pltpu.VMEM((CHUNK, DIM), jnp.float32),
            pltpu.VMEM((HC, DIM, BB), jnp.float32),
            pltpu.VMEM((HC, DIM, BB), jnp.float32),
            pltpu.VMEM((2, DIM), jnp.float32),
            pltpu.SemaphoreType.DMA,
            pltpu.SemaphoreType.DMA,
            pltpu.SemaphoreType.DMA,
            pltpu.SemaphoreType.DMA,
            pltpu.SemaphoreType.DMA,
        ],
        compiler_params=pltpu.CompilerParams(
            needs_layout_passes=False, use_tc_tiling_on_sc=False),
    )(_body)
    return f(idxes, table, beg_end)


def kernel(idxes, table, beg_end):
    return _run(idxes, table, beg_end).transpose(2, 0, 1)
